# baseline (speedup 1.0000x reference)
import functools

import jax
import jax.numpy as jnp
from jax import lax
from jax.experimental import pallas as pl
from jax.experimental.pallas import tpu as pltpu
from jax.experimental.pallas import tpu_sc as plsc

N = 10000
D = 128
E = 320000

NC = 2
NS = 16
NW = NC * NS
CH = 128
NCHUNK = 80
E_PAD = NW * NCHUNK * CH
N_PAD = 10240
RPT = N_PAD // NS

BLK = 1000
F32 = jnp.float32


@functools.cache
def _mesh():
    return plsc.VectorSubcoreMesh(core_axis_name="c", subcore_axis_name="s",
                                  num_cores=NC, num_subcores=NS)


def _sc_deg_body(dst_hbm, ones_hbm, zeros_hbm, out_hbm, dst_v, ones_v, shared):
    cid = lax.axis_index("c")
    sid = lax.axis_index("s")
    wid = sid * NC + cid
    pltpu.sync_copy(dst_hbm.at[wid], dst_v)
    pltpu.sync_copy(ones_hbm, ones_v)
    pltpu.sync_copy(zeros_hbm.at[pl.ds(sid * RPT, RPT)],
                    shared.at[pl.ds(sid * RPT, RPT)])
    plsc.subcore_barrier()

    def body(j, c):
        pltpu.sync_copy(ones_v, shared.at[dst_v.at[j]], add=True)
        return c

    lax.fori_loop(0, NCHUNK, body, 0)
    plsc.subcore_barrier()
    pltpu.sync_copy(shared.at[pl.ds(sid * RPT, RPT)],
                    out_hbm.at[cid, pl.ds(sid * RPT, RPT)])


@functools.cache
def _sc_deg():
    return pl.kernel(
        _sc_deg_body,
        out_type=jax.ShapeDtypeStruct((NC, N_PAD, D), F32),
        mesh=_mesh(),
        scratch_types=[
            pltpu.VMEM((NCHUNK, CH), jnp.int32),
            pltpu.VMEM((CH, D), F32),
            pltpu.VMEM_SHARED((N_PAD, D), F32),
        ],
    )


_HALF = NCHUNK // 2


def _sc_adj_body(h_hbm, src_hbm, dst_hbm, zeros_hbm, out_hbm,
                 src_v, dst_v, buf0, buf1, shared,
                 semg0, semg1, sems0, sems1):
    cid = lax.axis_index("c")
    sid = lax.axis_index("s")
    wid = sid * NC + cid
    pltpu.sync_copy(src_hbm.at[wid, pl.ds(0, _HALF)], src_v)
    pltpu.sync_copy(dst_hbm.at[wid, pl.ds(0, _HALF)], dst_v)
    pltpu.sync_copy(zeros_hbm.at[pl.ds(sid * RPT, RPT)],
                    shared.at[pl.ds(sid * RPT, RPT)])
    plsc.subcore_barrier()
    bufs = (buf0, buf1)
    semg = (semg0, semg1)
    sems = (sems0, sems1)

    for half in range(2):
        if half == 1:
            pltpu.sync_copy(src_hbm.at[wid, pl.ds(_HALF, _HALF)], src_v)
            pltpu.sync_copy(dst_hbm.at[wid, pl.ds(_HALF, _HALF)], dst_v)
        pltpu.async_copy(h_hbm.at[src_v.at[0]], buf0, semg0)
        pltpu.async_copy(h_hbm.at[src_v.at[1]], buf1, semg1)

        def body(i, c):
            for b in range(2):
                j = 2 * i + b
                pltpu.make_async_copy(
                    h_hbm.at[src_v.at[0]], bufs[b], semg[b]).wait()
                pltpu.async_copy(bufs[b], shared.at[dst_v.at[j]], sems[b],
                                 add=True)
            for b in range(2):
                j = 2 * i + b
                pltpu.make_async_copy(
                    bufs[b], shared.at[dst_v.at[0]], sems[b]).wait()
                pltpu.async_copy(h_hbm.at[src_v.at[j + 2]], bufs[b], semg[b])
            return c

        lax.fori_loop(0, _HALF // 2 - 1, body, 0)
        for b in range(2):
            j = _HALF - 2 + b
            pltpu.make_async_copy(
                h_hbm.at[src_v.at[0]], bufs[b], semg[b]).wait()
            pltpu.sync_copy(bufs[b], shared.at[dst_v.at[j]], add=True)

    plsc.subcore_barrier()
    pltpu.sync_copy(shared.at[pl.ds(sid * RPT, RPT)],
                    out_hbm.at[cid, pl.ds(sid * RPT, RPT)])


@functools.cache
def _sc_adj():
    return pl.kernel(
        _sc_adj_body,
        out_type=jax.ShapeDtypeStruct((NC, N_PAD, D), F32),
        mesh=_mesh(),
        scratch_types=[
            pltpu.VMEM((_HALF, CH), jnp.int32),
            pltpu.VMEM((_HALF, CH), jnp.int32),
            pltpu.VMEM((CH, D), F32),
            pltpu.VMEM((CH, D), F32),
            pltpu.VMEM_SHARED((N_PAD, D), F32),
            pltpu.SemaphoreType.DMA,
            pltpu.SemaphoreType.DMA,
            pltpu.SemaphoreType.DMA,
            pltpu.SemaphoreType.DMA,
        ],
    )


def _tc_prep_body(degp_ref, feat_ref, dinv_ref, x0s_ref):
    deg = degp_ref[0, :, 0:1] + degp_ref[1, :, 0:1]
    dinv = lax.rsqrt(jnp.maximum(deg, 1.0))
    dinv_b = jnp.broadcast_to(dinv, (BLK, D))
    dinv_ref[...] = dinv_b
    x0s_ref[...] = feat_ref[...] * dinv_b


_tc_prep = pl.pallas_call(
    _tc_prep_body,
    grid=(N // BLK,),
    in_specs=[
        pl.BlockSpec((NC, BLK, D), lambda i: (0, i, 0)),
        pl.BlockSpec((BLK, D), lambda i: (i, 0)),
    ],
    out_specs=[pl.BlockSpec((BLK, D), lambda i: (i, 0))] * 2,
    out_shape=[jax.ShapeDtypeStruct((N, D), F32)] * 2,
)


def _tc_mid_body(p_ref, dinv_ref, x1_ref, x1s_ref):
    dinv = dinv_ref[...]
    x1 = -(p_ref[0] + p_ref[1]) * dinv
    x1_ref[...] = x1
    x1s_ref[...] = x1 * dinv


_tc_mid = pl.pallas_call(
    _tc_mid_body,
    grid=(N // BLK,),
    in_specs=[
        pl.BlockSpec((NC, BLK, D), lambda i: (0, i, 0)),
        pl.BlockSpec((BLK, D), lambda i: (i, 0)),
    ],
    out_specs=[pl.BlockSpec((BLK, D), lambda i: (i, 0))] * 2,
    out_shape=[jax.ShapeDtypeStruct((N, D), F32)] * 2,
)


def _tc_comb_body(p_ref, dinv_ref, x0_ref, x1_ref, wt_ref, b_ref,
                  scale_ref, shift_ref, y_ref, ys_ref):
    dinv = dinv_ref[...]
    x0 = x0_ref[...]
    x1 = x1_ref[...]
    x2 = -2.0 * (p_ref[0] + p_ref[1]) * dinv - x0
    z = (jnp.dot(x0, wt_ref[0:D], preferred_element_type=F32)
         + jnp.dot(x1, wt_ref[D:2 * D], preferred_element_type=F32)
         + jnp.dot(x2, wt_ref[2 * D:3 * D], preferred_element_type=F32)
         + b_ref[...])
    y = jnp.maximum(z, 0.0) * scale_ref[...] + shift_ref[...]
    y_ref[...] = y
    ys_ref[...] = y * dinv


_tc_comb = pl.pallas_call(
    _tc_comb_body,
    grid=(N // BLK,),
    in_specs=[
        pl.BlockSpec((NC, BLK, D), lambda i: (0, i, 0)),
        pl.BlockSpec((BLK, D), lambda i: (i, 0)),
        pl.BlockSpec((BLK, D), lambda i: (i, 0)),
        pl.BlockSpec((BLK, D), lambda i: (i, 0)),
        pl.BlockSpec((3 * D, D), lambda i: (0, 0)),
        pl.BlockSpec((1, D), lambda i: (0, 0)),
        pl.BlockSpec((1, D), lambda i: (0, 0)),
        pl.BlockSpec((1, D), lambda i: (0, 0)),
    ],
    out_specs=[pl.BlockSpec((BLK, D), lambda i: (i, 0))] * 2,
    out_shape=[jax.ShapeDtypeStruct((N, D), F32)] * 2,
)


def _tc_final_body(p_ref, dinv_ref, x0_ref, x1_ref, wt_ref, b_ref,
                   wm1_ref, bm1_ref, wm2_ref, bm2_ref, out_ref):
    dinv = dinv_ref[...]
    x0 = x0_ref[...]
    x1 = x1_ref[...]
    x2 = -2.0 * (p_ref[0] + p_ref[1]) * dinv - x0
    z = (jnp.dot(x0, wt_ref[0:D], preferred_element_type=F32)
         + jnp.dot(x1, wt_ref[D:2 * D], preferred_element_type=F32)
         + jnp.dot(x2, wt_ref[2 * D:3 * D], preferred_element_type=F32)
         + b_ref[...])
    y = jnp.maximum(z, 0.0) + x0
    h = jnp.maximum(jnp.dot(y, wm1_ref[...], preferred_element_type=F32)
                    + bm1_ref[...], 0.0)
    out_ref[...] = (jnp.dot(h, wm2_ref[...], preferred_element_type=F32)
                    + bm2_ref[...])


_tc_final = pl.pallas_call(
    _tc_final_body,
    grid=(N // BLK,),
    in_specs=[
        pl.BlockSpec((NC, BLK, D), lambda i: (0, i, 0)),
        pl.BlockSpec((BLK, D), lambda i: (i, 0)),
        pl.BlockSpec((BLK, D), lambda i: (i, 0)),
        pl.BlockSpec((BLK, D), lambda i: (i, 0)),
        pl.BlockSpec((3 * D, D), lambda i: (0, 0)),
        pl.BlockSpec((1, D), lambda i: (0, 0)),
        pl.BlockSpec((D, D), lambda i: (0, 0)),
        pl.BlockSpec((1, D), lambda i: (0, 0)),
        pl.BlockSpec((D, D), lambda i: (0, 0)),
        pl.BlockSpec((1, D), lambda i: (0, 0)),
    ],
    out_specs=pl.BlockSpec((BLK, D), lambda i: (i, 0)),
    out_shape=jax.ShapeDtypeStruct((N, D), F32),
)


def kernel(features, edge_index, W_c1, b_c1, W_c2, b_c2, W_c3, b_c3,
           bn_gamma, bn_beta, W_m1, b_m1, W_m2, b_m2):
    pad = E_PAD - E
    src_p = jnp.concatenate([edge_index[0], jnp.zeros((pad,), jnp.int32)])
    dst_p = jnp.concatenate(
        [edge_index[1], jnp.full((pad,), N_PAD - 1, jnp.int32)])
    src3 = src_p.reshape(NW, NCHUNK, CH)
    dst3 = dst_p.reshape(NW, NCHUNK, CH)
    zeros128 = jnp.zeros((N_PAD, D), F32)
    ones128 = jnp.ones((CH, D), F32)

    degp = _sc_deg()(dst3, ones128, zeros128)
    dinv_b, x0s = _tc_prep(degp, features)

    one = jnp.ones((1, D), F32)
    zero = jnp.zeros((1, D), F32)
    gamma_p = (bn_gamma / jnp.sqrt(1.0 + 1e-5)).reshape(1, D)
    beta_p = bn_beta.reshape(1, D)

    def cheb(x, xs, wt, b2, scale, shift):
        p1 = _sc_adj()(xs, src3, dst3, zeros128)
        x1, x1s = _tc_mid(p1, dinv_b)
        p2 = _sc_adj()(x1s, src3, dst3, zeros128)
        return _tc_comb(p2, dinv_b, x, x1, wt, b2, scale, shift)

    wt1 = W_c1.T
    wt2 = W_c2.T
    x, xs = cheb(features, x0s, wt1, b_c1.reshape(1, D), gamma_p, beta_p)
    x, xs = cheb(x, xs, wt2, b_c2.reshape(1, D), one, zero)
    x, xs = cheb(x, xs, wt2, b_c2.reshape(1, D), one, zero)

    p1 = _sc_adj()(xs, src3, dst3, zeros128)
    x1, x1s = _tc_mid(p1, dinv_b)
    p2 = _sc_adj()(x1s, src3, dst3, zeros128)
    return _tc_final(p2, dinv_b, x, x1, W_c3.T, b_c3.reshape(1, D),
                     W_m1.T, b_m1.reshape(1, D), W_m2.T, b_m2.reshape(1, D))


# speedup vs baseline: 1.0403x; 1.0403x over previous
import functools

import jax
import jax.numpy as jnp
from jax import lax
from jax.experimental import pallas as pl
from jax.experimental.pallas import tpu as pltpu
from jax.experimental.pallas import tpu_sc as plsc

N = 10000
D = 128
E = 320000

NC = 2
NS = 16
NW = NC * NS
CH = 128
NCHUNK = 80
E_PAD = NW * NCHUNK * CH
N_PAD = 10240
RPT = N_PAD // NS

BLK = 1000
F32 = jnp.float32


@functools.cache
def _mesh():
    return plsc.VectorSubcoreMesh(core_axis_name="c", subcore_axis_name="s",
                                  num_cores=NC, num_subcores=NS)


def _sc_deg_body(dst_hbm, ones_hbm, zeros_hbm, out_hbm, dst_v, ones_v, shared):
    cid = lax.axis_index("c")
    sid = lax.axis_index("s")
    wid = sid * NC + cid
    pltpu.sync_copy(dst_hbm.at[wid], dst_v)
    pltpu.sync_copy(ones_hbm, ones_v)
    pltpu.sync_copy(zeros_hbm.at[pl.ds(sid * RPT, RPT)],
                    shared.at[pl.ds(sid * RPT, RPT)])
    plsc.subcore_barrier()

    def body(j, c):
        pltpu.sync_copy(ones_v, shared.at[dst_v.at[j]], add=True)
        return c

    lax.fori_loop(0, NCHUNK, body, 0)
    plsc.subcore_barrier()
    pltpu.sync_copy(shared.at[pl.ds(sid * RPT, RPT)],
                    out_hbm.at[cid, pl.ds(sid * RPT, RPT)])


@functools.cache
def _sc_deg():
    return pl.kernel(
        _sc_deg_body,
        out_type=jax.ShapeDtypeStruct((NC, N_PAD, D), F32),
        mesh=_mesh(),
        scratch_types=[
            pltpu.VMEM((NCHUNK, CH), jnp.int32),
            pltpu.VMEM((CH, D), F32),
            pltpu.VMEM_SHARED((N_PAD, D), F32),
        ],
    )


_HALF = NCHUNK // 2


def _sc_adj_body(h_hbm, src_hbm, dst_hbm, zeros_hbm, out_hbm,
                 src_v, dst_v, buf0, buf1, shared, semg0, semg1):
    cid = lax.axis_index("c")
    sid = lax.axis_index("s")
    wid = sid * NC + cid
    pltpu.sync_copy(src_hbm.at[wid, pl.ds(0, _HALF)], src_v)
    pltpu.sync_copy(dst_hbm.at[wid, pl.ds(0, _HALF)], dst_v)
    pltpu.sync_copy(zeros_hbm.at[pl.ds(sid * RPT, RPT)],
                    shared.at[pl.ds(sid * RPT, RPT)])
    plsc.subcore_barrier()
    bufs = (buf0, buf1)
    semg = (semg0, semg1)

    for half in range(2):
        if half == 1:
            pltpu.sync_copy(src_hbm.at[wid, pl.ds(_HALF, _HALF)], src_v)
            pltpu.sync_copy(dst_hbm.at[wid, pl.ds(_HALF, _HALF)], dst_v)
        pltpu.async_copy(h_hbm.at[src_v.at[0]], buf0, semg0)
        pltpu.async_copy(h_hbm.at[src_v.at[1]], buf1, semg1)

        def body(i, c):
            for b in range(2):
                j = 2 * i + b
                pltpu.make_async_copy(
                    h_hbm.at[src_v.at[0]], bufs[b], semg[b]).wait()
                pltpu.sync_copy(bufs[b], shared.at[dst_v.at[j]], add=True)
                pltpu.async_copy(h_hbm.at[src_v.at[j + 2]], bufs[b], semg[b])
            return c

        lax.fori_loop(0, _HALF // 2 - 1, body, 0)
        for b in range(2):
            j = _HALF - 2 + b
            pltpu.make_async_copy(
                h_hbm.at[src_v.at[0]], bufs[b], semg[b]).wait()
            pltpu.sync_copy(bufs[b], shared.at[dst_v.at[j]], add=True)

    plsc.subcore_barrier()
    pltpu.sync_copy(shared.at[pl.ds(sid * RPT, RPT)],
                    out_hbm.at[cid, pl.ds(sid * RPT, RPT)])


@functools.cache
def _sc_adj():
    return pl.kernel(
        _sc_adj_body,
        out_type=jax.ShapeDtypeStruct((NC, N_PAD, D), F32),
        mesh=_mesh(),
        scratch_types=[
            pltpu.VMEM((_HALF, CH), jnp.int32),
            pltpu.VMEM((_HALF, CH), jnp.int32),
            pltpu.VMEM((CH, D), F32),
            pltpu.VMEM((CH, D), F32),
            pltpu.VMEM_SHARED((N_PAD, D), F32),
            pltpu.SemaphoreType.DMA,
            pltpu.SemaphoreType.DMA,
        ],
    )


def _tc_prep_body(degp_ref, feat_ref, dinv_ref, x0s_ref):
    deg = degp_ref[0, :, 0:1] + degp_ref[1, :, 0:1]
    dinv = lax.rsqrt(jnp.maximum(deg, 1.0))
    dinv_b = jnp.broadcast_to(dinv, (BLK, D))
    dinv_ref[...] = dinv_b
    x0s_ref[...] = feat_ref[...] * dinv_b


_tc_prep = pl.pallas_call(
    _tc_prep_body,
    grid=(N // BLK,),
    in_specs=[
        pl.BlockSpec((NC, BLK, D), lambda i: (0, i, 0)),
        pl.BlockSpec((BLK, D), lambda i: (i, 0)),
    ],
    out_specs=[pl.BlockSpec((BLK, D), lambda i: (i, 0))] * 2,
    out_shape=[jax.ShapeDtypeStruct((N, D), F32)] * 2,
)


def _tc_mid_body(p_ref, dinv_ref, x1_ref, x1s_ref):
    dinv = dinv_ref[...]
    x1 = -(p_ref[0] + p_ref[1]) * dinv
    x1_ref[...] = x1
    x1s_ref[...] = x1 * dinv


_tc_mid = pl.pallas_call(
    _tc_mid_body,
    grid=(N // BLK,),
    in_specs=[
        pl.BlockSpec((NC, BLK, D), lambda i: (0, i, 0)),
        pl.BlockSpec((BLK, D), lambda i: (i, 0)),
    ],
    out_specs=[pl.BlockSpec((BLK, D), lambda i: (i, 0))] * 2,
    out_shape=[jax.ShapeDtypeStruct((N, D), F32)] * 2,
)


def _tc_comb_body(p_ref, dinv_ref, x0_ref, x1_ref, wt_ref, b_ref,
                  scale_ref, shift_ref, y_ref, ys_ref):
    dinv = dinv_ref[...]
    x0 = x0_ref[...]
    x1 = x1_ref[...]
    x2 = -2.0 * (p_ref[0] + p_ref[1]) * dinv - x0
    z = (jnp.dot(x0, wt_ref[0:D], preferred_element_type=F32)
         + jnp.dot(x1, wt_ref[D:2 * D], preferred_element_type=F32)
         + jnp.dot(x2, wt_ref[2 * D:3 * D], preferred_element_type=F32)
         + b_ref[...])
    y = jnp.maximum(z, 0.0) * scale_ref[...] + shift_ref[...]
    y_ref[...] = y
    ys_ref[...] = y * dinv


_tc_comb = pl.pallas_call(
    _tc_comb_body,
    grid=(N // BLK,),
    in_specs=[
        pl.BlockSpec((NC, BLK, D), lambda i: (0, i, 0)),
        pl.BlockSpec((BLK, D), lambda i: (i, 0)),
        pl.BlockSpec((BLK, D), lambda i: (i, 0)),
        pl.BlockSpec((BLK, D), lambda i: (i, 0)),
        pl.BlockSpec((3 * D, D), lambda i: (0, 0)),
        pl.BlockSpec((1, D), lambda i: (0, 0)),
        pl.BlockSpec((1, D), lambda i: (0, 0)),
        pl.BlockSpec((1, D), lambda i: (0, 0)),
    ],
    out_specs=[pl.BlockSpec((BLK, D), lambda i: (i, 0))] * 2,
    out_shape=[jax.ShapeDtypeStruct((N, D), F32)] * 2,
)


def _tc_final_body(p_ref, dinv_ref, x0_ref, x1_ref, wt_ref, b_ref,
                   wm1_ref, bm1_ref, wm2_ref, bm2_ref, out_ref):
    dinv = dinv_ref[...]
    x0 = x0_ref[...]
    x1 = x1_ref[...]
    x2 = -2.0 * (p_ref[0] + p_ref[1]) * dinv - x0
    z = (jnp.dot(x0, wt_ref[0:D], preferred_element_type=F32)
         + jnp.dot(x1, wt_ref[D:2 * D], preferred_element_type=F32)
         + jnp.dot(x2, wt_ref[2 * D:3 * D], preferred_element_type=F32)
         + b_ref[...])
    y = jnp.maximum(z, 0.0) + x0
    h = jnp.maximum(jnp.dot(y, wm1_ref[...], preferred_element_type=F32)
                    + bm1_ref[...], 0.0)
    out_ref[...] = (jnp.dot(h, wm2_ref[...], preferred_element_type=F32)
                    + bm2_ref[...])


_tc_final = pl.pallas_call(
    _tc_final_body,
    grid=(N // BLK,),
    in_specs=[
        pl.BlockSpec((NC, BLK, D), lambda i: (0, i, 0)),
        pl.BlockSpec((BLK, D), lambda i: (i, 0)),
        pl.BlockSpec((BLK, D), lambda i: (i, 0)),
        pl.BlockSpec((BLK, D), lambda i: (i, 0)),
        pl.BlockSpec((3 * D, D), lambda i: (0, 0)),
        pl.BlockSpec((1, D), lambda i: (0, 0)),
        pl.BlockSpec((D, D), lambda i: (0, 0)),
        pl.BlockSpec((1, D), lambda i: (0, 0)),
        pl.BlockSpec((D, D), lambda i: (0, 0)),
        pl.BlockSpec((1, D), lambda i: (0, 0)),
    ],
    out_specs=pl.BlockSpec((BLK, D), lambda i: (i, 0)),
    out_shape=jax.ShapeDtypeStruct((N, D), F32),
)


def kernel(features, edge_index, W_c1, b_c1, W_c2, b_c2, W_c3, b_c3,
           bn_gamma, bn_beta, W_m1, b_m1, W_m2, b_m2):
    pad = E_PAD - E
    src_p = jnp.concatenate([edge_index[0], jnp.zeros((pad,), jnp.int32)])
    dst_p = jnp.concatenate(
        [edge_index[1], jnp.full((pad,), N_PAD - 1, jnp.int32)])
    src3 = src_p.reshape(NW, NCHUNK, CH)
    dst3 = dst_p.reshape(NW, NCHUNK, CH)
    zeros128 = jnp.zeros((N_PAD, D), F32)
    ones128 = jnp.ones((CH, D), F32)

    degp = _sc_deg()(dst3, ones128, zeros128)
    dinv_b, x0s = _tc_prep(degp, features)

    one = jnp.ones((1, D), F32)
    zero = jnp.zeros((1, D), F32)
    gamma_p = (bn_gamma / jnp.sqrt(1.0 + 1e-5)).reshape(1, D)
    beta_p = bn_beta.reshape(1, D)

    def cheb(x, xs, wt, b2, scale, shift):
        p1 = _sc_adj()(xs, src3, dst3, zeros128)
        x1, x1s = _tc_mid(p1, dinv_b)
        p2 = _sc_adj()(x1s, src3, dst3, zeros128)
        return _tc_comb(p2, dinv_b, x, x1, wt, b2, scale, shift)

    wt1 = W_c1.T
    wt2 = W_c2.T
    x, xs = cheb(features, x0s, wt1, b_c1.reshape(1, D), gamma_p, beta_p)
    x, xs = cheb(x, xs, wt2, b_c2.reshape(1, D), one, zero)
    x, xs = cheb(x, xs, wt2, b_c2.reshape(1, D), one, zero)

    p1 = _sc_adj()(xs, src3, dst3, zeros128)
    x1, x1s = _tc_mid(p1, dinv_b)
    p2 = _sc_adj()(x1s, src3, dst3, zeros128)
    return _tc_final(p2, dinv_b, x, x1, W_c3.T, b_c3.reshape(1, D),
                     W_m1.T, b_m1.reshape(1, D), W_m2.T, b_m2.reshape(1, D))
